# baseline (speedup 1.0000x reference)
import dataclasses
import functools

import jax
import jax.numpy as jnp
from jax import lax
from jax.experimental import pallas as pl
from jax.experimental.pallas import tpu as pltpu
from jax.experimental.pallas import tpu_sc as plsc

N = 10000
E = 320000
D = 128
B = 128
NC, NS = 2, 16
NW = NC * NS
ROWS = (E + B - 1) // B
BPW = (-(-ROWS // NW) + 7) // 8 * 8
ROWS_PAD = BPW * NW
NPAD = -(-(N + 1) // (NS * 8)) * NS * 8
RPT = NPAD // NS
HW = NPAD // 2
RB = 8
CHUNKS = BPW // RB


def _sc_scatter(featx, src2d, dst2d, z128):
  mesh = plsc.VectorSubcoreMesh(
      core_axis_name="c", subcore_axis_name="s", num_cores=NC, num_subcores=NS)
  cp = pltpu.CompilerParams()
  if "needs_layout_passes" in pltpu.CompilerParams.__dataclass_fields__:
    cp = dataclasses.replace(cp, needs_layout_passes=False)

  @functools.partial(
      pl.kernel,
      compiler_params=cp,
      out_type=[
          jax.ShapeDtypeStruct((NC, NPAD, D), jnp.float32),
          jax.ShapeDtypeStruct((NC, NS, HW), jnp.int32),
      ],
      mesh=mesh,
      scratch_types=[
          pltpu.VMEM((RB, B), jnp.int32),
          pltpu.VMEM((RB, B), jnp.int32),
          pltpu.VMEM((B, D), jnp.float32),
          pltpu.VMEM((B, D), jnp.float32),
          pltpu.VMEM((HW,), jnp.int32),
          pltpu.VMEM_SHARED((NPAD, D), jnp.float32),
          pltpu.SemaphoreType.DMA,
          pltpu.SemaphoreType.DMA,
      ],
  )
  def k(feat_hbm, src_hbm, dst_hbm, z128_hbm,
        psum_hbm, pdeg_hbm, src_v, dst_v, gb0, gb1, hist_v, ssum,
        gsem, scsem):
    c = lax.axis_index("c")
    s = lax.axis_index("s")
    wid = c * NS + s
    row0 = wid * BPW
    srow = s * RPT
    lane = lax.iota(jnp.int32, 16)
    gb = (gb0, gb1)

    pltpu.sync_copy(z128_hbm, ssum.at[pl.ds(srow, RPT)])

    def zro(i, carry):
      hist_v[pl.ds(i * 16, 16)] = jnp.zeros((16,), jnp.int32)
      return carry

    lax.fori_loop(0, HW // 16, zro, 0)
    plsc.subcore_barrier()

    def hist_batch(j):
      def hst(g, carry2):
        dvec = dst_v[j, pl.ds(g * 16, 16)]
        for l in range(16):
          d = dvec[l]
          wi = lax.shift_right_logical(d, 1)
          base = jnp.bitwise_and(wi, -16)
          off = wi - base
          addv = lax.shift_left(1, jnp.bitwise_and(d, 1) * 16)
          w = hist_v[pl.ds(base, 16)]
          hist_v[pl.ds(base, 16)] = w + jnp.where(lane == off, addv, 0)
        return carry2

      lax.fori_loop(0, B // 16, hst, 0)

    def gather(j, buf):
      return pltpu.async_copy(feat_hbm.at[src_v.at[j]], buf, gsem)

    def scatter(j, buf):
      return pltpu.async_copy(buf, ssum.at[dst_v.at[j]], scsem, add=True)

    def wait_sc():
      pltpu.make_async_copy(gb1, ssum.at[dst_v.at[RB - 1]], scsem).wait()

    def chunk_body(ci, first):
      pltpu.sync_copy(src_hbm.at[pl.ds(row0 + ci * RB, RB)], src_v)
      g0 = gather(0, gb0)
      if not first:
        wait_sc()
      pltpu.sync_copy(dst_hbm.at[pl.ds(row0 + ci * RB, RB)], dst_v)
      g0.wait()
      gnext = gather(1, gb1)
      sc = scatter(0, gb0)
      hist_batch(0)
      for j in range(1, RB):
        gnext.wait()
        sc.wait()
        if j < RB - 1:
          gnext = gather(j + 1, gb[(j + 1) % 2])
        sc = scatter(j, gb[j % 2])
        hist_batch(j)

    chunk_body(0, True)

    def chunk(ci, carry):
      chunk_body(ci, False)
      return carry

    lax.fori_loop(1, CHUNKS, chunk, 0)
    wait_sc()
    plsc.subcore_barrier()

    pltpu.sync_copy(ssum.at[pl.ds(srow, RPT)], psum_hbm.at[c, pl.ds(srow, RPT)])
    pltpu.sync_copy(hist_v, pdeg_hbm.at[c, s])

  return k(featx, src2d, dst2d, z128)


def _combine_body(ps_ref, pd_ref, o_ref):
  ssum = ps_ref[0] + ps_ref[1]
  deg = jnp.sum(pd_ref[...].astype(jnp.float32), axis=(0, 1))
  rdeg = 1.0 / jnp.maximum(deg, 1.0)
  o_ref[...] = ssum * rdeg[:, None]


def _combine(psum, deg16):
  return pl.pallas_call(
      _combine_body,
      out_shape=jax.ShapeDtypeStruct((NPAD, D), jnp.float32),
  )(psum, deg16)


@jax.jit
def kernel(feat, edge_index):
  src = edge_index[0].astype(jnp.int32)
  dst = edge_index[1].astype(jnp.int32)
  pad = ROWS_PAD * B - E
  featx = jnp.concatenate([feat, jnp.zeros((16, D), feat.dtype)])
  pad_src = N + jnp.arange(pad, dtype=jnp.int32) % 16
  pad_dst = N + jnp.arange(pad, dtype=jnp.int32) % (NPAD - N)
  src2d = jnp.concatenate([src, pad_src]).reshape(ROWS_PAD, B)
  dst2d = jnp.concatenate([dst, pad_dst]).reshape(ROWS_PAD, B)
  z128 = jnp.zeros((RPT, D), jnp.float32)
  psum, pdeg = _sc_scatter(featx, src2d, dst2d, z128)
  deg16 = lax.bitcast_convert_type(pdeg, jnp.int16).reshape(NC, NS, NPAD)
  return _combine(psum, deg16)[:N]


# speedup vs baseline: 1.2124x; 1.2124x over previous
import dataclasses
import functools

import jax
import jax.numpy as jnp
from jax import lax
from jax.experimental import pallas as pl
from jax.experimental.pallas import tpu as pltpu
from jax.experimental.pallas import tpu_sc as plsc

N = 10000
E = 320000
D = 128
B = 128
NC, NS = 2, 16
NW = NC * NS
ROWS = (E + B - 1) // B
BPW = (-(-ROWS // NW) + 7) // 8 * 8
ROWS_PAD = BPW * NW
NPAD = -(-(N + 1) // (NS * 8)) * NS * 8
RPT = NPAD // NS
HW = NPAD // 2
RB = 8
CHUNKS = BPW // RB


def _sc_scatter(featx, src2d, dst2d, z128):
  mesh = plsc.VectorSubcoreMesh(
      core_axis_name="c", subcore_axis_name="s", num_cores=NC, num_subcores=NS)
  cp = pltpu.CompilerParams()
  if "needs_layout_passes" in pltpu.CompilerParams.__dataclass_fields__:
    cp = dataclasses.replace(cp, needs_layout_passes=False)

  @functools.partial(
      pl.kernel,
      compiler_params=cp,
      out_type=[
          jax.ShapeDtypeStruct((NC, NPAD, D), jnp.float32),
          jax.ShapeDtypeStruct((NC, NS, HW), jnp.int32),
      ],
      mesh=mesh,
      scratch_types=[
          pltpu.VMEM((RB, B), jnp.int32),
          pltpu.VMEM((RB, B), jnp.int32),
          pltpu.VMEM((B, D), jnp.float32),
          pltpu.VMEM((B, D), jnp.float32),
          pltpu.VMEM((HW,), jnp.int32),
          pltpu.VMEM_SHARED((NPAD, D), jnp.float32),
          pltpu.SemaphoreType.DMA,
          pltpu.SemaphoreType.DMA,
      ],
  )
  def k(feat_hbm, src_hbm, dst_hbm, z128_hbm,
        psum_hbm, pdeg_hbm, src_v, dst_v, gb0, gb1, hist_v, ssum,
        gsem, scsem):
    c = lax.axis_index("c")
    s = lax.axis_index("s")
    wid = c * NS + s
    row0 = wid * BPW
    srow = s * RPT
    lane = lax.iota(jnp.int32, 16)
    gb = (gb0, gb1)

    pltpu.sync_copy(z128_hbm, ssum.at[pl.ds(srow, RPT)])

    def zro(i, carry):
      hist_v[pl.ds(i * 16, 16)] = jnp.zeros((16,), jnp.int32)
      return carry

    lax.fori_loop(0, HW // 16, zro, 0)
    plsc.subcore_barrier()

    def hist_batch(j):
      def hst(g, carry2):
        dvec = dst_v[j, pl.ds(g * 16, 16)]
        for l in range(16):
          d = dvec[l]
          ge = (d >= HW).astype(jnp.int32)
          wi = d - ge * HW
          base = jnp.bitwise_and(wi, -16)
          off = wi - base
          addv = lax.shift_left(1, ge * 16)
          w = hist_v[pl.ds(base, 16)]
          hist_v[pl.ds(base, 16)] = w + jnp.where(lane == off, addv, 0)
        return carry2

      lax.fori_loop(0, B // 16, hst, 0)

    def gather(j, buf):
      return pltpu.async_copy(feat_hbm.at[src_v.at[j]], buf, gsem)

    def scatter(j, buf):
      return pltpu.async_copy(buf, ssum.at[dst_v.at[j]], scsem, add=True)

    def wait_sc():
      pltpu.make_async_copy(gb1, ssum.at[dst_v.at[RB - 1]], scsem).wait()

    def chunk_body(ci, first):
      pltpu.sync_copy(src_hbm.at[pl.ds(row0 + ci * RB, RB)], src_v)
      g0 = gather(0, gb0)
      if not first:
        wait_sc()
      pltpu.sync_copy(dst_hbm.at[pl.ds(row0 + ci * RB, RB)], dst_v)
      g0.wait()
      gnext = gather(1, gb1)
      sc = scatter(0, gb0)
      hist_batch(0)
      for j in range(1, RB):
        gnext.wait()
        sc.wait()
        if j < RB - 1:
          gnext = gather(j + 1, gb[(j + 1) % 2])
        sc = scatter(j, gb[j % 2])
        hist_batch(j)

    chunk_body(0, True)

    def chunk(ci, carry):
      chunk_body(ci, False)
      return carry

    lax.fori_loop(1, CHUNKS, chunk, 0)
    wait_sc()
    plsc.subcore_barrier()

    pltpu.sync_copy(ssum.at[pl.ds(srow, RPT)], psum_hbm.at[c, pl.ds(srow, RPT)])
    pltpu.sync_copy(hist_v, pdeg_hbm.at[c, s])

  return k(featx, src2d, dst2d, z128)


def _combine_body(ps_ref, pd_ref, o_ref):
  ssum = ps_ref[0] + ps_ref[1]
  pd = pd_ref[...]
  dlow = jnp.sum(jnp.bitwise_and(pd, 0xFFFF), axis=(0, 1)).astype(jnp.float32)
  dhigh = jnp.sum(lax.shift_right_logical(pd, 16), axis=(0, 1)).astype(jnp.float32)
  rlow = 1.0 / jnp.maximum(dlow, 1.0)
  rhigh = 1.0 / jnp.maximum(dhigh, 1.0)
  rdeg = jnp.concatenate([rlow, rhigh])
  o_ref[...] = (ssum * rdeg[:, None])[:N]


def _combine(psum, pdeg):
  return pl.pallas_call(
      _combine_body,
      out_shape=jax.ShapeDtypeStruct((N, D), jnp.float32),
  )(psum, pdeg)


@jax.jit
def kernel(feat, edge_index):
  src = edge_index[0].astype(jnp.int32)
  dst = edge_index[1].astype(jnp.int32)
  pad = ROWS_PAD * B - E
  pad_src = jnp.arange(pad, dtype=jnp.int32) % 16
  pad_dst = N + jnp.arange(pad, dtype=jnp.int32) % (NPAD - N)
  src2d = jnp.concatenate([src, pad_src]).reshape(ROWS_PAD, B)
  dst2d = jnp.concatenate([dst, pad_dst]).reshape(ROWS_PAD, B)
  z128 = jnp.zeros((RPT, D), jnp.float32)
  psum, pdeg = _sc_scatter(feat, src2d, dst2d, z128)
  return _combine(psum, pdeg)


# speedup vs baseline: 1.2402x; 1.0229x over previous
import dataclasses
import functools

import jax
import jax.numpy as jnp
from jax import lax
from jax.experimental import pallas as pl
from jax.experimental.pallas import tpu as pltpu
from jax.experimental.pallas import tpu_sc as plsc

N = 10000
E = 320000
D = 128
B = 128
NC, NS = 2, 16
NW = NC * NS
ROWS = (E + B - 1) // B
BPW = (-(-ROWS // NW) + 7) // 8 * 8
ROWS_PAD = BPW * NW
NPAD = -(-(N + 1) // (NS * 8)) * NS * 8
RPT = NPAD // NS
HW = NPAD // 2
RB = 8
CHUNKS = BPW // RB
ROWS_ALL = ROWS_PAD + RB


def _sc_scatter(featx, src2d, dst2d, z128):
  mesh = plsc.VectorSubcoreMesh(
      core_axis_name="c", subcore_axis_name="s", num_cores=NC, num_subcores=NS)
  cp = pltpu.CompilerParams()
  if "needs_layout_passes" in pltpu.CompilerParams.__dataclass_fields__:
    cp = dataclasses.replace(cp, needs_layout_passes=False)

  @functools.partial(
      pl.kernel,
      compiler_params=cp,
      out_type=[
          jax.ShapeDtypeStruct((NC, NPAD, D), jnp.float32),
          jax.ShapeDtypeStruct((NC, NS, HW), jnp.int32),
      ],
      mesh=mesh,
      scratch_types=[
          pltpu.VMEM((RB, B), jnp.int32),
          pltpu.VMEM((RB, B), jnp.int32),
          pltpu.VMEM((B, D), jnp.float32),
          pltpu.VMEM((B, D), jnp.float32),
          pltpu.VMEM((HW,), jnp.int32),
          pltpu.VMEM_SHARED((NPAD, D), jnp.float32),
          pltpu.SemaphoreType.DMA,
          pltpu.SemaphoreType.DMA,
          pltpu.SemaphoreType.DMA,
      ],
  )
  def k(feat_hbm, src_hbm, dst_hbm, z128_hbm,
        psum_hbm, pdeg_hbm, src_v, dst_v, gb0, gb1, hist_v, ssum,
        gsem, scsem, rsem):
    c = lax.axis_index("c")
    s = lax.axis_index("s")
    wid = c * NS + s
    row0 = wid * BPW
    srow = s * RPT
    lane = lax.iota(jnp.int32, 16)
    gb = (gb0, gb1)

    pltpu.async_copy(src_hbm.at[pl.ds(row0, RB)], src_v, rsem)

    pltpu.sync_copy(z128_hbm, ssum.at[pl.ds(srow, RPT)])

    def zro(i, carry):
      hist_v[pl.ds(i * 16, 16)] = jnp.zeros((16,), jnp.int32)
      return carry

    lax.fori_loop(0, HW // 16, zro, 0)
    plsc.subcore_barrier()

    def hist_batch(j):
      def hst(g, carry2):
        dvec = dst_v[j, pl.ds(g * 16, 16)]
        for l in range(16):
          d = dvec[l]
          ge = (d >= HW).astype(jnp.int32)
          wi = d - ge * HW
          base = jnp.bitwise_and(wi, -16)
          off = wi - base
          addv = lax.shift_left(1, ge * 16)
          w = hist_v[pl.ds(base, 16)]
          hist_v[pl.ds(base, 16)] = w + jnp.where(lane == off, addv, 0)
        return carry2

      lax.fori_loop(0, B // 16, hst, 0)

    def gather(j, buf):
      return pltpu.async_copy(feat_hbm.at[src_v.at[j]], buf, gsem)

    def scatter(j, buf):
      return pltpu.async_copy(buf, ssum.at[dst_v.at[j]], scsem, add=True)

    def wait_sc():
      pltpu.make_async_copy(gb1, ssum.at[dst_v.at[RB - 1]], scsem).wait()

    def wait_src_prefetch():
      pltpu.make_async_copy(src_hbm.at[pl.ds(0, RB)], src_v, rsem).wait()

    def chunk_body(ci, first):
      wait_src_prefetch()
      g0 = gather(0, gb0)
      if not first:
        wait_sc()
      pltpu.sync_copy(dst_hbm.at[pl.ds(row0 + ci * RB, RB)], dst_v)
      g0.wait()
      gnext = gather(1, gb1)
      sc = scatter(0, gb0)
      hist_batch(0)
      for j in range(1, RB):
        gnext.wait()
        sc.wait()
        if j < RB - 1:
          gnext = gather(j + 1, gb[(j + 1) % 2])
        else:
          pltpu.async_copy(
              src_hbm.at[pl.ds(row0 + (ci + 1) * RB, RB)], src_v, rsem)
        sc = scatter(j, gb[j % 2])
        hist_batch(j)

    chunk_body(0, True)

    def chunk(ci, carry):
      chunk_body(ci, False)
      return carry

    lax.fori_loop(1, CHUNKS, chunk, 0)
    wait_sc()
    wait_src_prefetch()
    plsc.subcore_barrier()

    pltpu.sync_copy(ssum.at[pl.ds(srow, RPT)], psum_hbm.at[c, pl.ds(srow, RPT)])
    pltpu.sync_copy(hist_v, pdeg_hbm.at[c, s])

  return k(featx, src2d, dst2d, z128)


def _combine_body(ps_ref, pd_ref, o_ref):
  ssum = ps_ref[0] + ps_ref[1]
  pd = pd_ref[...]
  dlow = jnp.sum(jnp.bitwise_and(pd, 0xFFFF), axis=(0, 1)).astype(jnp.float32)
  dhigh = jnp.sum(lax.shift_right_logical(pd, 16), axis=(0, 1)).astype(jnp.float32)
  rlow = 1.0 / jnp.maximum(dlow, 1.0)
  rhigh = 1.0 / jnp.maximum(dhigh, 1.0)
  rdeg = jnp.concatenate([rlow, rhigh])
  o_ref[...] = (ssum * rdeg[:, None])[:N]


def _combine(psum, pdeg):
  return pl.pallas_call(
      _combine_body,
      out_shape=jax.ShapeDtypeStruct((N, D), jnp.float32),
  )(psum, pdeg)


@jax.jit
def kernel(feat, edge_index):
  src = edge_index[0].astype(jnp.int32)
  dst = edge_index[1].astype(jnp.int32)
  pad = ROWS_ALL * B - E
  pad_src = jnp.arange(pad, dtype=jnp.int32) % 16
  pad_dst = N + jnp.arange(pad, dtype=jnp.int32) % (NPAD - N)
  src2d = jnp.concatenate([src, pad_src]).reshape(ROWS_ALL, B)
  dst2d = jnp.concatenate([dst, pad_dst]).reshape(ROWS_ALL, B)
  z128 = jnp.zeros((RPT, D), jnp.float32)
  psum, pdeg = _sc_scatter(feat, src2d, dst2d, z128)
  return _combine(psum, pdeg)


# speedup vs baseline: 1.3037x; 1.0512x over previous
import dataclasses
import functools

import jax
import jax.numpy as jnp
from jax import lax
from jax.experimental import pallas as pl
from jax.experimental.pallas import tpu as pltpu
from jax.experimental.pallas import tpu_sc as plsc

N = 10000
E = 320000
D = 128
B = 128
NC, NS = 2, 16
NW = NC * NS
ROWS = (E + B - 1) // B
BPW = (-(-ROWS // NW) + 7) // 8 * 8
ROWS_PAD = BPW * NW
NPAD = -(-(N + 1) // (NS * 8)) * NS * 8
RPT = NPAD // NS
HW = NPAD // 2
RB = 8
CHUNKS = BPW // RB
ROWS_ALL = ROWS_PAD + RB


def _sc_scatter(feat, edges3d, z128):
  mesh = plsc.VectorSubcoreMesh(
      core_axis_name="c", subcore_axis_name="s", num_cores=NC, num_subcores=NS)
  cp = pltpu.CompilerParams()
  if "needs_layout_passes" in pltpu.CompilerParams.__dataclass_fields__:
    cp = dataclasses.replace(cp, needs_layout_passes=False)

  @functools.partial(
      pl.kernel,
      compiler_params=cp,
      out_type=[
          jax.ShapeDtypeStruct((NC, NPAD, D), jnp.float32),
          jax.ShapeDtypeStruct((NC, NS, HW), jnp.int32),
      ],
      mesh=mesh,
      scratch_types=[
          pltpu.VMEM((RB, B), jnp.int32),
          pltpu.VMEM((RB, B), jnp.int32),
          pltpu.VMEM((B, D), jnp.float32),
          pltpu.VMEM((B, D), jnp.float32),
          pltpu.VMEM((HW,), jnp.int32),
          pltpu.VMEM_SHARED((NPAD, D), jnp.float32),
          pltpu.SemaphoreType.DMA,
          pltpu.SemaphoreType.DMA,
          pltpu.SemaphoreType.DMA,
      ],
  )
  def k(feat_hbm, e_hbm, z128_hbm,
        psum_hbm, pdeg_hbm, src_v, dst_v, gb0, gb1, hist_v, ssum,
        gsem, scsem, rsem):
    c = lax.axis_index("c")
    s = lax.axis_index("s")
    wid = c * NS + s
    row0 = wid * BPW
    srow = s * RPT
    lane = lax.iota(jnp.int32, 16)
    gb = (gb0, gb1)

    pltpu.async_copy(e_hbm.at[0, pl.ds(row0, RB)], src_v, rsem)

    pltpu.sync_copy(z128_hbm, ssum.at[pl.ds(srow, RPT)])

    def zro(i, carry):
      hist_v[pl.ds(i * 16, 16)] = jnp.zeros((16,), jnp.int32)
      return carry

    lax.fori_loop(0, HW // 16, zro, 0)
    plsc.subcore_barrier()

    def hist_batch(j):
      def hst(g, carry2):
        dvec = dst_v[j, pl.ds(g * 16, 16)]
        for l in range(16):
          d = dvec[l]
          ge = (d >= HW).astype(jnp.int32)
          wi = d - ge * HW
          base = jnp.bitwise_and(wi, -16)
          off = wi - base
          addv = lax.shift_left(1, ge * 16)
          w = hist_v[pl.ds(base, 16)]
          hist_v[pl.ds(base, 16)] = w + jnp.where(lane == off, addv, 0)
        return carry2

      lax.fori_loop(0, B // 16, hst, 0)

    def gather(j, buf):
      return pltpu.async_copy(feat_hbm.at[src_v.at[j]], buf, gsem)

    def scatter(j, buf):
      return pltpu.async_copy(buf, ssum.at[dst_v.at[j]], scsem, add=True)

    def wait_sc():
      pltpu.make_async_copy(gb1, ssum.at[dst_v.at[RB - 1]], scsem).wait()

    def wait_src_prefetch():
      pltpu.make_async_copy(e_hbm.at[0, pl.ds(0, RB)], src_v, rsem).wait()

    def chunk_body(ci, first):
      wait_src_prefetch()
      g0 = gather(0, gb0)
      if not first:
        wait_sc()
      pltpu.sync_copy(e_hbm.at[1, pl.ds(row0 + ci * RB, RB)], dst_v)
      g0.wait()
      gnext = gather(1, gb1)
      sc = scatter(0, gb0)
      hist_batch(0)
      for j in range(1, RB):
        gnext.wait()
        sc.wait()
        if j < RB - 1:
          gnext = gather(j + 1, gb[(j + 1) % 2])
        else:
          pltpu.async_copy(
              e_hbm.at[0, pl.ds(row0 + (ci + 1) * RB, RB)], src_v, rsem)
        sc = scatter(j, gb[j % 2])
        hist_batch(j)

    chunk_body(0, True)

    def chunk(ci, carry):
      chunk_body(ci, False)
      return carry

    lax.fori_loop(1, CHUNKS, chunk, 0)
    wait_sc()
    wait_src_prefetch()
    plsc.subcore_barrier()

    pltpu.sync_copy(ssum.at[pl.ds(srow, RPT)], psum_hbm.at[c, pl.ds(srow, RPT)])
    pltpu.sync_copy(hist_v, pdeg_hbm.at[c, s])

  return k(feat, edges3d, z128)


def _combine_body(ps_ref, pd_ref, o_ref):
  ssum = ps_ref[0] + ps_ref[1]
  pd = pd_ref[...]
  dlow = jnp.sum(jnp.bitwise_and(pd, 0xFFFF), axis=(0, 1)).astype(jnp.float32)
  dhigh = jnp.sum(lax.shift_right_logical(pd, 16), axis=(0, 1)).astype(jnp.float32)
  rlow = 1.0 / jnp.maximum(dlow, 1.0)
  rhigh = 1.0 / jnp.maximum(dhigh, 1.0)
  rdeg = jnp.concatenate([rlow, rhigh])
  o_ref[...] = (ssum * rdeg[:, None])[:N]


def _combine(psum, pdeg):
  return pl.pallas_call(
      _combine_body,
      out_shape=jax.ShapeDtypeStruct((N, D), jnp.float32),
  )(psum, pdeg)


@jax.jit
def kernel(feat, edge_index):
  e3 = edge_index.astype(jnp.int32).reshape(2, ROWS, B)
  pad = ROWS_ALL * B - E
  pad_src = (jnp.arange(pad, dtype=jnp.int32) % 16).reshape(-1, B)
  pad_dst = (N + jnp.arange(pad, dtype=jnp.int32) % (NPAD - N)).reshape(-1, B)
  edges3d = jnp.concatenate([e3, jnp.stack([pad_src, pad_dst])], axis=1)
  z128 = jnp.zeros((RPT, D), jnp.float32)
  psum, pdeg = _sc_scatter(feat, edges3d, z128)
  return _combine(psum, pdeg)


# speedup vs baseline: 1.3574x; 1.0412x over previous
import dataclasses
import functools

import jax
import jax.numpy as jnp
from jax import lax
from jax.experimental import pallas as pl
from jax.experimental.pallas import tpu as pltpu
from jax.experimental.pallas import tpu_sc as plsc

N = 10000
E = 320000
D = 128
B = 128
NC, NS = 2, 16
NW = NC * NS
ROWS = (E + B - 1) // B
BPW = (-(-ROWS // NW) + 7) // 8 * 8
ROWS_PAD = BPW * NW
NPAD = -(-(N + 1) // (NS * 8)) * NS * 8
RPT = NPAD // NS
HW = NPAD // 2
RB = 8
CHUNKS = BPW // RB
ROWS_ALL = ROWS_PAD + RB


def _sc_scatter(feat, edges3d, z128):
  mesh = plsc.VectorSubcoreMesh(
      core_axis_name="c", subcore_axis_name="s", num_cores=NC, num_subcores=NS)
  cp = pltpu.CompilerParams()
  if "needs_layout_passes" in pltpu.CompilerParams.__dataclass_fields__:
    cp = dataclasses.replace(cp, needs_layout_passes=False)

  @functools.partial(
      pl.kernel,
      compiler_params=cp,
      out_type=[
          jax.ShapeDtypeStruct((NC, NPAD, D), jnp.float32),
          jax.ShapeDtypeStruct((NC, NS, HW), jnp.int32),
      ],
      mesh=mesh,
      scratch_types=[
          pltpu.VMEM((RB, B), jnp.int32),
          pltpu.VMEM((RB, B), jnp.int32),
          pltpu.VMEM((B, D), jnp.float32),
          pltpu.VMEM((B, D), jnp.float32),
          pltpu.VMEM((HW,), jnp.int32),
          pltpu.VMEM_SHARED((NPAD, D), jnp.float32),
          pltpu.SemaphoreType.DMA,
          pltpu.SemaphoreType.DMA,
          pltpu.SemaphoreType.DMA,
      ],
  )
  def k(feat_hbm, e_hbm, z128_hbm,
        psum_hbm, pdeg_hbm, src_v, dst_v, gb0, gb1, hist_v, ssum,
        gsem, scsem, rsem):
    c = lax.axis_index("c")
    s = lax.axis_index("s")
    wid = c * NS + s
    row0 = wid * BPW
    srow = s * RPT
    lane = lax.iota(jnp.int32, 16)
    gb = (gb0, gb1)

    pltpu.async_copy(e_hbm.at[0, pl.ds(row0, RB)], src_v, rsem)

    pltpu.sync_copy(z128_hbm, ssum.at[pl.ds(srow, RPT)])

    def zro(i, carry):
      hist_v[pl.ds(i * 16, 16)] = jnp.zeros((16,), jnp.int32)
      return carry

    lax.fori_loop(0, HW // 16, zro, 0)
    plsc.subcore_barrier()

    def hist_batch(j):
      def hst(g, carry2):
        dvec = dst_v[j, pl.ds(g * 16, 16)]
        lo = dvec < HW
        wvec = jnp.where(lo, dvec, dvec - HW)
        for half, addshift in ((lo, 0), (jnp.logical_not(lo), 16)):
          cnt, last = plsc.scan_count(dvec, mask=half)
          cur = plsc.load_gather(hist_v, [wvec])
          upd = cur + lax.shift_left(cnt, addshift)
          plsc.store_scatter(hist_v, [wvec], upd, mask=last)
        return carry2

      lax.fori_loop(0, B // 16, hst, 0)

    def gather(j, buf):
      return pltpu.async_copy(feat_hbm.at[src_v.at[j]], buf, gsem)

    def scatter(j, buf):
      return pltpu.async_copy(buf, ssum.at[dst_v.at[j]], scsem, add=True)

    def wait_sc():
      pltpu.make_async_copy(gb1, ssum.at[dst_v.at[RB - 1]], scsem).wait()

    def wait_src_prefetch():
      pltpu.make_async_copy(e_hbm.at[0, pl.ds(0, RB)], src_v, rsem).wait()

    def chunk_body(ci, first):
      wait_src_prefetch()
      g0 = gather(0, gb0)
      if not first:
        wait_sc()
      pltpu.sync_copy(e_hbm.at[1, pl.ds(row0 + ci * RB, RB)], dst_v)
      g0.wait()
      gnext = gather(1, gb1)
      sc = scatter(0, gb0)
      hist_batch(0)
      for j in range(1, RB):
        gnext.wait()
        sc.wait()
        if j < RB - 1:
          gnext = gather(j + 1, gb[(j + 1) % 2])
        else:
          pltpu.async_copy(
              e_hbm.at[0, pl.ds(row0 + (ci + 1) * RB, RB)], src_v, rsem)
        sc = scatter(j, gb[j % 2])
        hist_batch(j)

    chunk_body(0, True)

    def chunk(ci, carry):
      chunk_body(ci, False)
      return carry

    lax.fori_loop(1, CHUNKS, chunk, 0)
    wait_sc()
    wait_src_prefetch()
    plsc.subcore_barrier()

    pltpu.sync_copy(ssum.at[pl.ds(srow, RPT)], psum_hbm.at[c, pl.ds(srow, RPT)])
    pltpu.sync_copy(hist_v, pdeg_hbm.at[c, s])

  return k(feat, edges3d, z128)


def _combine_body(ps_ref, pd_ref, o_ref):
  ssum = ps_ref[0] + ps_ref[1]
  pd = pd_ref[...]
  dlow = jnp.sum(jnp.bitwise_and(pd, 0xFFFF), axis=(0, 1)).astype(jnp.float32)
  dhigh = jnp.sum(lax.shift_right_logical(pd, 16), axis=(0, 1)).astype(jnp.float32)
  rlow = 1.0 / jnp.maximum(dlow, 1.0)
  rhigh = 1.0 / jnp.maximum(dhigh, 1.0)
  rdeg = jnp.concatenate([rlow, rhigh])
  o_ref[...] = (ssum * rdeg[:, None])[:N]


def _combine(psum, pdeg):
  return pl.pallas_call(
      _combine_body,
      out_shape=jax.ShapeDtypeStruct((N, D), jnp.float32),
  )(psum, pdeg)


@jax.jit
def kernel(feat, edge_index):
  e3 = edge_index.astype(jnp.int32).reshape(2, ROWS, B)
  pad = ROWS_ALL * B - E
  pad_src = (jnp.arange(pad, dtype=jnp.int32) % 16).reshape(-1, B)
  pad_dst = (N + jnp.arange(pad, dtype=jnp.int32) % (NPAD - N)).reshape(-1, B)
  edges3d = jnp.concatenate([e3, jnp.stack([pad_src, pad_dst])], axis=1)
  z128 = jnp.zeros((RPT, D), jnp.float32)
  psum, pdeg = _sc_scatter(feat, edges3d, z128)
  return _combine(psum, pdeg)
